# baseline (speedup 1.0000x reference)
import functools

import jax
import jax.numpy as jnp
from jax import lax
from jax.experimental import pallas as pl
from jax.experimental.pallas import tpu as pltpu
from jax.experimental.pallas import tpu_sc as plsc

_G = 12
_CHUNK = 16384
_BUFS = 4
_RA = 2


def _tc_body(x_ref, r_ref, xo_ref, ro_ref):
    xo_ref[...] = x_ref[...]
    ro_ref[...] = r_ref[...]


def _ring_copy(ops, bufs, sem_rd, sem_wr):
    k = len(ops)
    nb = len(bufs)
    ra = min(_RA, nb - 1)
    rds = [None] * k
    wrs = [None] * k
    for b in range(min(ra, k)):
        rds[b] = pltpu.async_copy(ops[b][0], bufs[b % nb], sem_rd)
    for i in range(k):
        j = i + ra
        if j < k:
            w = j - nb
            if w >= 0:
                wrs[w].wait()
            rds[j] = pltpu.async_copy(ops[j][0], bufs[j % nb], sem_rd)
        rds[i].wait()
        wrs[i] = pltpu.async_copy(bufs[i % nb], ops[i][1], sem_wr)
    for i in range(max(0, k - nb), k):
        if wrs[i] is not None:
            wrs[i].wait()


def _make_sc_dists_copy(n):
    info = plsc.get_sparse_core_info()
    nc, ns = info.num_cores, info.num_subcores
    nw = nc * ns
    seg = n // nw
    steps = seg // _CHUNK
    mesh = plsc.VectorSubcoreMesh(core_axis_name="c", subcore_axis_name="s")

    @functools.partial(
        pl.kernel,
        mesh=mesh,
        out_type=jax.ShapeDtypeStruct((n,), jnp.float32),
        scratch_types=(
            [pltpu.VMEM((_CHUNK,), jnp.float32)] * _BUFS
            + [pltpu.SemaphoreType.DMA, pltpu.SemaphoreType.DMA]
        ),
    )
    def k(d_hbm, do_hbm, *rest):
        bufs = list(rest[:_BUFS])
        sem_rd, sem_wr = rest[_BUFS], rest[_BUFS + 1]
        wid = lax.axis_index("s") * nc + lax.axis_index("c")
        base = wid * seg
        ops = []
        for c in range(steps):
            sl = pl.ds(base + c * _CHUNK, _CHUNK)
            ops.append((d_hbm.at[sl], do_hbm.at[sl]))
        _ring_copy(ops, bufs, sem_rd, sem_wr)

    return k


def kernel(sampled_point_xyz, sampled_point_ray_direction, sampled_point_distance):
    n = sampled_point_xyz.shape[0]
    b = 349568
    xt = sampled_point_xyz.T
    rt = sampled_point_ray_direction.T
    dists = _make_sc_dists_copy(n)(sampled_point_distance)
    pos_t, ray_t = pl.pallas_call(
        _tc_body,
        grid=(_G,),
        in_specs=[
            pl.BlockSpec((3, b), lambda i: (0, i)),
            pl.BlockSpec((3, b), lambda i: (0, i)),
        ],
        out_specs=[
            pl.BlockSpec((3, b), lambda i: (0, i)),
            pl.BlockSpec((3, b), lambda i: (0, i)),
        ],
        out_shape=[
            jax.ShapeDtypeStruct((3, n), jnp.float32),
            jax.ShapeDtypeStruct((3, n), jnp.float32),
        ],
    )(xt, rt)
    return (pos_t.T, ray_t.T, dists)


# speedup vs baseline: 1.1711x; 1.1711x over previous
import jax
import jax.numpy as jnp
from jax.experimental import pallas as pl
from jax.experimental.pallas import tpu as pltpu

_G = 10


def _copy_body(x_ref, r_ref, d_ref, xo_ref, ro_ref, do_ref):
    xo_ref[...] = x_ref[...]
    ro_ref[...] = r_ref[...]
    do_ref[...] = d_ref[...]


def kernel(sampled_point_xyz, sampled_point_ray_direction, sampled_point_distance):
    n = sampled_point_xyz.shape[0]
    b = 419456
    bd = 419840
    xt = sampled_point_xyz.T
    rt = sampled_point_ray_direction.T
    pos_t, ray_t, dists = pl.pallas_call(
        _copy_body,
        grid=(_G,),
        in_specs=[
            pl.BlockSpec((3, b), lambda i: (0, i)),
            pl.BlockSpec((3, b), lambda i: (0, i)),
            pl.BlockSpec((bd,), lambda i: (i,)),
        ],
        out_specs=[
            pl.BlockSpec((3, b), lambda i: (0, i)),
            pl.BlockSpec((3, b), lambda i: (0, i)),
            pl.BlockSpec((bd,), lambda i: (i,)),
        ],
        out_shape=[
            jax.ShapeDtypeStruct((3, n), jnp.float32),
            jax.ShapeDtypeStruct((3, n), jnp.float32),
            jax.ShapeDtypeStruct((n,), jnp.float32),
        ],
    )(xt, rt, sampled_point_distance)
    return (pos_t.T, ray_t.T, dists)


# speedup vs baseline: 1.1761x; 1.0043x over previous
import jax
import jax.numpy as jnp
from jax.experimental import pallas as pl
from jax.experimental.pallas import tpu as pltpu

_G = 11


def _copy_body(x_ref, r_ref, d_ref, xo_ref, ro_ref, do_ref):
    xo_ref[...] = x_ref[...]
    ro_ref[...] = r_ref[...]
    do_ref[...] = d_ref[...]


def kernel(sampled_point_xyz, sampled_point_ray_direction, sampled_point_distance):
    n = sampled_point_xyz.shape[0]
    b = 381312
    bd = 381952
    xt = sampled_point_xyz.T
    rt = sampled_point_ray_direction.T
    pos_t, ray_t, dists = pl.pallas_call(
        _copy_body,
        grid=(_G,),
        in_specs=[
            pl.BlockSpec((3, b), lambda i: (0, i)),
            pl.BlockSpec((3, b), lambda i: (0, i)),
            pl.BlockSpec((bd,), lambda i: (i,)),
        ],
        out_specs=[
            pl.BlockSpec((3, b), lambda i: (0, i)),
            pl.BlockSpec((3, b), lambda i: (0, i)),
            pl.BlockSpec((bd,), lambda i: (i,)),
        ],
        out_shape=[
            jax.ShapeDtypeStruct((3, n), jnp.float32),
            jax.ShapeDtypeStruct((3, n), jnp.float32),
            jax.ShapeDtypeStruct((n,), jnp.float32),
        ],
    )(xt, rt, sampled_point_distance)
    return (pos_t.T, ray_t.T, dists)


# speedup vs baseline: 1.1766x; 1.0005x over previous
import jax
import jax.numpy as jnp
from jax.experimental import pallas as pl
from jax.experimental.pallas import tpu as pltpu

_G = 12


def _copy_body(x_ref, r_ref, d_ref, xo_ref, ro_ref, do_ref):
    xo_ref[...] = x_ref[...]
    ro_ref[...] = r_ref[...]
    do_ref[...] = d_ref[...]


def kernel(sampled_point_xyz, sampled_point_ray_direction, sampled_point_distance):
    n = sampled_point_xyz.shape[0]
    b = 349568
    bd = 350208
    xt = sampled_point_xyz.T
    rt = sampled_point_ray_direction.T
    pos_t, ray_t, dists = pl.pallas_call(
        _copy_body,
        grid=(_G,),
        in_specs=[
            pl.BlockSpec((3, b), lambda i: (0, i)),
            pl.BlockSpec((3, b), lambda i: (0, i)),
            pl.BlockSpec((bd,), lambda i: (i,)),
        ],
        out_specs=[
            pl.BlockSpec((3, b), lambda i: (0, i)),
            pl.BlockSpec((3, b), lambda i: (0, i)),
            pl.BlockSpec((bd,), lambda i: (i,)),
        ],
        out_shape=[
            jax.ShapeDtypeStruct((3, n), jnp.float32),
            jax.ShapeDtypeStruct((3, n), jnp.float32),
            jax.ShapeDtypeStruct((n,), jnp.float32),
        ],
    )(xt, rt, sampled_point_distance)
    return (pos_t.T, ray_t.T, dists)
